# Initial kernel scaffold; baseline (speedup 1.0000x reference)
import jax
import jax.numpy as jnp
from jax.experimental import pallas as pl


def kernel(tensor, indices, W_kernel, W_bias, Uf_kernel, Uiuo_kernel):
    raise NotImplementedError("write your pallas kernel here")


# speedup vs baseline: 1.1371x; 1.1371x over previous
import functools

import jax
import jax.numpy as jnp
from jax.experimental import pallas as pl

DIN = 256
D = 256


def _wx_body(x_ref, w_ref, b_ref, o_ref):
    o_ref[...] = (
        jnp.dot(x_ref[...], w_ref[...], preferred_element_type=jnp.float32)
        + b_ref[...]
    )


def _wx_matmul(x2, W_kernel, W_bias):
    M = x2.shape[0]
    BM = 1024
    grid = (M // BM,)
    return pl.pallas_call(
        _wx_body,
        grid=grid,
        in_specs=[
            pl.BlockSpec((BM, DIN), lambda i: (i, 0)),
            pl.BlockSpec((DIN, 4 * D), lambda i: (0, 0)),
            pl.BlockSpec((1, 4 * D), lambda i: (0, 0)),
        ],
        out_specs=pl.BlockSpec((BM, 4 * D), lambda i: (i, 0)),
        out_shape=jax.ShapeDtypeStruct((M, 4 * D), jnp.float32),
    )(x2, W_kernel, W_bias.reshape(1, 4 * D))


def kernel(tensor, indices, W_kernel, W_bias, Uf_kernel, Uiuo_kernel):
    L, N, _ = tensor.shape
    d = D
    Wx = _wx_matmul(tensor.reshape(L * N, DIN), W_kernel, W_bias)
    Wx = Wx.reshape(L, N, 4 * d)

    res_h, res_c = [], []
    tab = None
    for t in range(L):
        Wx_t = Wx[t]
        Wf_x = Wx_t[:, :d]
        Wi_x = Wx_t[:, d:2 * d]
        Wu_x = Wx_t[:, 2 * d:3 * d]
        Wo_x = Wx_t[:, 3 * d:]
        if t == 0:
            h_sum = jnp.zeros((N, d), jnp.float32)
            fco = jnp.zeros((N, d), jnp.float32)
        else:
            safe = jnp.maximum(indices[t], 0)
            g = jnp.take(tab, safe, axis=0)
            h_g = g[..., :d]
            c_g = g[..., d:2 * d]
            hU_g = g[..., 2 * d:]
            h_sum = jnp.sum(h_g, axis=1)
            f = jax.nn.sigmoid(Wf_x[:, None, :] + hU_g)
            fco = jnp.sum(f * c_g, axis=1)
        iuo = h_sum @ Uiuo_kernel
        i = jax.nn.sigmoid(iuo[:, :d] + Wi_x)
        u = jnp.tanh(iuo[:, d:2 * d] + Wu_x)
        o = jax.nn.sigmoid(iuo[:, 2 * d:] + Wo_x)
        new_c = i * u + fco
        new_h = o * jnp.tanh(new_c)
        hU = new_h @ Uf_kernel
        tab = jnp.concatenate(
            [jnp.zeros((1, 3 * d), jnp.float32),
             jnp.concatenate([new_h, new_c, hU], axis=1)],
            axis=0)
        res_h.append(new_h)
        res_c.append(new_c)
    return (jnp.stack(res_h), jnp.stack(res_c))
